# Initial kernel scaffold; baseline (speedup 1.0000x reference)
import jax
import jax.numpy as jnp
from jax.experimental import pallas as pl


def kernel(x, edge_index, block_index, edge_imaginary_index, batch, node_embedding, params):
    raise NotImplementedError("write your pallas kernel here")


# speedup vs baseline: 9.7171x; 9.7171x over previous
import functools

import jax
import jax.numpy as jnp
from jax import lax
from jax.experimental import pallas as pl
from jax.experimental.pallas import tpu as pltpu
from jax.experimental.pallas import tpu_sc as plsc

_HIGH = lax.Precision.HIGHEST

H = 128
NPAD = 10240
DROWS = NPAD // H
APAD = 10368
SROWS = APAD // 16
ZW = 72
EW = 128


def _dot(a, b):
    return jnp.dot(a, b, precision=_HIGH, preferred_element_type=jnp.float32)


def _enc_body(x_ref, e_ref, w1x, w1e, b1, w2, b2, w3, b3, o_ref):
    t = _dot(x_ref[...], w1x[...]) + _dot(e_ref[...], w1e[...]) + b1[...]
    t = jnp.maximum(t, 0.0)
    t = jnp.maximum(_dot(t, w2[...]) + b2[...], 0.0)
    o_ref[...] = _dot(t, w3[...]) + b3[...]


def _encoder(x, emb, p, blk=1000):
    n, dx = x.shape
    de = emb.shape[1]
    return pl.pallas_call(
        _enc_body,
        grid=(n // blk,),
        in_specs=[
            pl.BlockSpec((blk, dx), lambda i: (i, 0)),
            pl.BlockSpec((blk, de), lambda i: (i, 0)),
            pl.BlockSpec((dx, H), lambda i: (0, 0)),
            pl.BlockSpec((de, H), lambda i: (0, 0)),
            pl.BlockSpec((1, H), lambda i: (0, 0)),
            pl.BlockSpec((H, H), lambda i: (0, 0)),
            pl.BlockSpec((1, H), lambda i: (0, 0)),
            pl.BlockSpec((H, H), lambda i: (0, 0)),
            pl.BlockSpec((1, H), lambda i: (0, 0)),
        ],
        out_specs=pl.BlockSpec((blk, H), lambda i: (i, 0)),
        out_shape=jax.ShapeDtypeStruct((n, H), jnp.float32),
    )(x, emb, p['W1'][:dx], p['W1'][dx:], p['b1'].reshape(1, H),
      p['W2'], p['b2'].reshape(1, H), p['W3'], p['b3'].reshape(1, H))


def _lr_body(h_ref, wl, bl, wr, br, xl_ref, xr_ref):
    hh = h_ref[...]
    xl_ref[...] = _dot(hh, wl[...]) + bl[...]
    xr_ref[...] = _dot(hh, wr[...]) + br[...]


def _lr(h, p, blk=1000):
    n = h.shape[0]
    return pl.pallas_call(
        _lr_body,
        grid=(n // blk,),
        in_specs=[
            pl.BlockSpec((blk, H), lambda i: (i, 0)),
            pl.BlockSpec((H, H), lambda i: (0, 0)),
            pl.BlockSpec((1, H), lambda i: (0, 0)),
            pl.BlockSpec((H, H), lambda i: (0, 0)),
            pl.BlockSpec((1, H), lambda i: (0, 0)),
        ],
        out_specs=[
            pl.BlockSpec((blk, H), lambda i: (i, 0)),
            pl.BlockSpec((blk, H), lambda i: (i, 0)),
        ],
        out_shape=[
            jax.ShapeDtypeStruct((n, H), jnp.float32),
            jax.ShapeDtypeStruct((n, H), jnp.float32),
        ],
    )(h, p['Wl'], p['bl'].reshape(1, H), p['Wr'], p['br'].reshape(1, H))


def _edge_body(gl_ref, gr_ref, dst_ref, att_ref, o_ref, *, blk, e_true):
    i = pl.program_id(0)
    gl = gl_ref[...]
    z = gl + gr_ref[...]
    z = jnp.maximum(z, 0.2 * z)
    s = jnp.sum(z * att_ref[...], axis=1, keepdims=True)
    rows = lax.broadcasted_iota(jnp.int32, (blk, 1), 0) + i * blk
    w = jnp.where(rows < e_true, jnp.exp(s), 0.0)
    o_ref[0] = w * gl
    lanes = lax.broadcasted_iota(jnp.int32, (blk, H), 1)
    o_ref[1] = jnp.where(lanes == dst_ref[...] % H, w, 0.0)


def _edge(gl, gr, dst_col, att, e_true, blk=2048):
    epad = gl.shape[0]
    body = functools.partial(_edge_body, blk=blk, e_true=e_true)
    return pl.pallas_call(
        body,
        grid=(epad // blk,),
        in_specs=[
            pl.BlockSpec((blk, H), lambda i: (i, 0)),
            pl.BlockSpec((blk, H), lambda i: (i, 0)),
            pl.BlockSpec((blk, 1), lambda i: (i, 0)),
            pl.BlockSpec((1, H), lambda i: (0, 0)),
        ],
        out_specs=pl.BlockSpec((2, blk, H), lambda i: (0, i, 0)),
        out_shape=jax.ShapeDtypeStruct((2, epad, H), jnp.float32),
    )(gl, gr, dst_col, att)


def _comb_body(p_ref, den_ref, bias, o_ref):
    num = p_ref[0] + p_ref[1]
    o_ref[...] = num / (den_ref[...] + 1e-16) + bias[...]


def _combine(parts, den_col, bias, n, blk=1024):
    nblk = -(-n // blk)
    return pl.pallas_call(
        _comb_body,
        grid=(nblk,),
        in_specs=[
            pl.BlockSpec((2, blk, H), lambda i: (0, i, 0)),
            pl.BlockSpec((blk, 1), lambda i: (i, 0)),
            pl.BlockSpec((1, H), lambda i: (0, 0)),
        ],
        out_specs=pl.BlockSpec((blk, H), lambda i: (i, 0)),
        out_shape=jax.ShapeDtypeStruct((n, H), jnp.float32),
    )(parts, den_col, bias)


def _dec_e_body(ga_ref, gb_ref, w1a, w1b, b1, w2, b2, w3, b3, o_ref):
    t = _dot(ga_ref[...], w1a[...]) + _dot(gb_ref[...], w1b[...]) + b1[...]
    t = jnp.maximum(t, 0.0)
    t = jnp.maximum(_dot(t, w2[...]) + b2[...], 0.0)
    o_ref[...] = jax.nn.sigmoid(_dot(t, w3[...]) + b3[...])


def _dec_e(g, p, m, blk=2048):
    nb = m // blk
    return pl.pallas_call(
        _dec_e_body,
        grid=(nb,),
        in_specs=[
            pl.BlockSpec((blk, H), lambda i: (i, 0)),
            pl.BlockSpec((blk, H), lambda i, nb=nb: (i + nb, 0)),
            pl.BlockSpec((H, H), lambda i: (0, 0)),
            pl.BlockSpec((H, H), lambda i: (0, 0)),
            pl.BlockSpec((1, H), lambda i: (0, 0)),
            pl.BlockSpec((H, H), lambda i: (0, 0)),
            pl.BlockSpec((1, H), lambda i: (0, 0)),
            pl.BlockSpec((H, 1), lambda i: (0, 0)),
            pl.BlockSpec((1, 1), lambda i: (0, 0)),
        ],
        out_specs=pl.BlockSpec((blk, 1), lambda i: (i, 0)),
        out_shape=jax.ShapeDtypeStruct((m, 1), jnp.float32),
    )(g, g, p['W1'][:H], p['W1'][H:], p['b1'].reshape(1, H),
      p['W2'], p['b2'].reshape(1, H), p['W3'], p['b3'].reshape(1, 1))


def _dec_n_body(g_ref, w1, b1, w2, b2, w3, b3, o_ref):
    t = jnp.maximum(_dot(g_ref[...], w1[...]) + b1[...], 0.0)
    t = jnp.maximum(_dot(t, w2[...]) + b2[...], 0.0)
    o_ref[...] = _dot(t, w3[...]) + b3[...]


def _dec_n(g, p, row0, nrows):
    out_dim = p['W3'].shape[1]
    blk0 = row0 // nrows
    return pl.pallas_call(
        _dec_n_body,
        grid=(1,),
        in_specs=[
            pl.BlockSpec((nrows, H), lambda i, blk0=blk0: (blk0, 0)),
            pl.BlockSpec((H, H), lambda i: (0, 0)),
            pl.BlockSpec((1, H), lambda i: (0, 0)),
            pl.BlockSpec((H, H), lambda i: (0, 0)),
            pl.BlockSpec((1, H), lambda i: (0, 0)),
            pl.BlockSpec((H, out_dim), lambda i: (0, 0)),
            pl.BlockSpec((1, out_dim), lambda i: (0, 0)),
        ],
        out_specs=pl.BlockSpec((nrows, out_dim), lambda i: (i, 0)),
        out_shape=jax.ShapeDtypeStruct((nrows, out_dim), jnp.float32),
    )(g, p['W1'], p['b1'].reshape(1, H), p['W2'], p['b2'].reshape(1, H),
      p['W3'], p['b3'].reshape(1, out_dim))


def _sc_mesh():
    return plsc.VectorSubcoreMesh(core_axis_name="c", subcore_axis_name="s")


def _sc_gather(table, idx):
    m = idx.shape[0]
    d = table.shape[1]
    idx2 = idx.reshape(1, m)

    @functools.partial(
        pl.kernel,
        out_type=jax.ShapeDtypeStruct((m, d), table.dtype),
        mesh=_sc_mesh())
    def k(x_hbm, i_hbm, o_hbm):
        def body(i_vmem, o_vmem):
            pltpu.sync_copy(x_hbm.at[i_vmem.at[0]], o_vmem)

        pltpu.emit_pipeline(
            body,
            grid=(m // EW,),
            in_specs=[pl.BlockSpec((1, EW), lambda i: (0, i))],
            out_specs=[pl.BlockSpec((EW, d), lambda i: (i, 0))],
            core_axis_name=("c", "s"),
            dimension_semantics=(pltpu.PARALLEL,),
        )(i_hbm, o_hbm)

    return k(table, idx2)


def _sc_gather2(xl, xr, isrc, idst):
    m = isrc.shape[0]
    d = xl.shape[1]
    is2 = isrc.reshape(1, m)
    id2 = idst.reshape(1, m)

    @functools.partial(
        pl.kernel,
        out_type=[jax.ShapeDtypeStruct((m, d), jnp.float32),
                  jax.ShapeDtypeStruct((m, d), jnp.float32)],
        mesh=_sc_mesh())
    def k(xl_hbm, xr_hbm, is_hbm, id_hbm, gl_hbm, gr_hbm):
        def body(is_v, id_v, gl_v, gr_v):
            pltpu.sync_copy(xl_hbm.at[is_v.at[0]], gl_v)
            pltpu.sync_copy(xr_hbm.at[id_v.at[0]], gr_v)

        pltpu.emit_pipeline(
            body,
            grid=(m // EW,),
            in_specs=[pl.BlockSpec((1, EW), lambda i: (0, i)),
                      pl.BlockSpec((1, EW), lambda i: (0, i))],
            out_specs=[pl.BlockSpec((EW, d), lambda i: (i, 0)),
                       pl.BlockSpec((EW, d), lambda i: (i, 0))],
            core_axis_name=("c", "s"),
            dimension_semantics=(pltpu.PARALLEL,),
        )(is_hbm, id_hbm, gl_hbm, gr_hbm)

    return k(xl, xr, is2, id2)


def _sc_scatter(wall, iall, zstripe):
    m = wall.shape[0]
    iall2 = iall.reshape(1, m)

    @functools.partial(
        pl.kernel,
        out_type=jax.ShapeDtypeStruct((2, APAD, H), jnp.float32),
        mesh=_sc_mesh(),
        scratch_types=[pltpu.VMEM_SHARED((APAD, H), jnp.float32)])
    def k(w_hbm, i_hbm, z_hbm, o_hbm, acc):
        cid = lax.axis_index("c")
        sid = lax.axis_index("s")

        @pl.loop(0, SROWS, step=ZW)
        def _(j):
            pltpu.sync_copy(z_hbm, acc.at[pl.ds(sid * SROWS + j, ZW)])
        plsc.subcore_barrier()

        def body(w_vmem, i_vmem):
            pltpu.sync_copy(w_vmem, acc.at[i_vmem.at[0]], add=True)

        pltpu.emit_pipeline(
            body,
            grid=(m // EW,),
            in_specs=[pl.BlockSpec((EW, H), lambda i: (i, 0)),
                      pl.BlockSpec((1, EW), lambda i: (0, i))],
            out_specs=[],
            core_axis_name=("c", "s"),
            dimension_semantics=(pltpu.PARALLEL,),
        )(w_hbm, i_hbm)
        plsc.subcore_barrier()

        @pl.loop(0, SROWS, step=ZW)
        def _(j):
            pltpu.sync_copy(acc.at[pl.ds(sid * SROWS + j, ZW)],
                            o_hbm.at[cid, pl.ds(sid * SROWS + j, ZW)])

    return k(wall, iall2, zstripe)


def kernel(x, edge_index, block_index, edge_imaginary_index, batch,
           node_embedding, params):
    n = x.shape[0]
    e = edge_index.shape[1]
    p = params

    emb = node_embedding[-n:]
    h = _encoder(x, emb, p['enc'])

    e_true = e + n
    epad = -(-e_true // (32 * EW)) * (32 * EW)
    padz = jnp.zeros((epad - e_true,), jnp.int32)
    sl = jnp.arange(n, dtype=jnp.int32)
    srce = jnp.concatenate([edge_index[0], sl, padz])
    dste = jnp.concatenate([edge_index[1], sl, padz])
    dst_col = dste.reshape(epad, 1)
    iall = jnp.concatenate([dste, NPAD + dste // H])
    zstripe = jnp.zeros((ZW, H), jnp.float32)

    for i in range(len(p['gat'])):
        gp = p['gat'][i]
        xl, xr = _lr(h, gp)
        gl, gr = _sc_gather2(xl, xr, srce, dste)
        wboth = _edge(gl, gr, dst_col, gp['att'].reshape(1, H), e_true)
        parts = _sc_scatter(wboth.reshape(2 * epad, H), iall, zstripe)
        den_col = (parts[0, NPAD:NPAD + DROWS] +
                   parts[1, NPAD:NPAD + DROWS]).reshape(NPAD, 1)
        h = _combine(parts, den_col, gp['bias'].reshape(1, H), n)

    m_img = edge_imaginary_index.shape[1]
    nb = block_index.shape[0]
    gidx = jnp.concatenate(
        [edge_imaginary_index[0], edge_imaginary_index[1], block_index])
    g = _sc_gather(h, gidx)
    edges_prob = _dec_e(g, p['dec_e'], m_img)
    nodes_out = _dec_n(g, p['dec_n'], 2 * m_img, nb)
    return (nodes_out, edges_prob)
